# Initial kernel scaffold; baseline (speedup 1.0000x reference)
import functools

import jax
import jax.numpy as jnp
from jax import lax
from jax.experimental import pallas as pl
from jax.experimental.pallas import tpu as pltpu
from jax.experimental.pallas import tpu_sc as plsc

N_BINS = 15
N = 262144
C = 100
R = 1024
NB = N // R
HIST = N_BINS * 128
NW = 32
CHUNK = N // NW
LANES = 16


def _tc_main(x_ref, lab_ref, conf_ref, flat_ref, labmax_ref):
    i = pl.program_id(0)
    x = x_ref[...]
    lab = lab_ref[0]

    m = jnp.max(x, axis=1, keepdims=True)
    e = jnp.exp(x - m)
    s = jnp.sum(e, axis=1, keepdims=True)
    sm = e / s

    bi = jnp.clip(jnp.ceil(sm * float(N_BINS)) - 1.0, 0.0, float(N_BINS - 1))

    rows = [
        jnp.sum(jnp.where(bi == float(b), sm, 0.0), axis=0, keepdims=True)
        for b in range(N_BINS)
    ]
    part = jnp.concatenate(rows, axis=0)

    onehot = lax.broadcasted_iota(jnp.int32, (R, C), 1) == lab
    conf_lab = jnp.sum(jnp.where(onehot, sm, 0.0), axis=1, keepdims=True)
    bl = jnp.clip(jnp.ceil(conf_lab * float(N_BINS)) - 1.0, 0.0,
                  float(N_BINS - 1)).astype(jnp.int32)
    flat_ref[0] = bl * 128 + lab

    lm = jnp.max(lab, axis=0, keepdims=True)

    @pl.when(i == 0)
    def _():
        conf_ref[...] = part
        labmax_ref[...] = lm

    @pl.when(i > 0)
    def _():
        conf_ref[...] += part
        labmax_ref[...] = jnp.maximum(labmax_ref[...], lm)


def _sc_hist(ids_hbm, out_hbm, ids_v, hist_v):
    info = plsc.get_sparse_core_info()
    wid = lax.axis_index("s") * info.num_cores + lax.axis_index("c")
    pltpu.sync_copy(ids_hbm.at[pl.ds(wid * CHUNK, CHUNK)], ids_v)

    zeros16 = jnp.zeros((LANES,), jnp.float32)
    ones16 = jnp.ones((LANES,), jnp.float32)
    lane_off = lax.iota(jnp.int32, LANES) * HIST

    def zero_body(j, _):
        base = j * (8 * LANES)
        for u in range(8):
            hist_v[pl.ds(base + u * LANES, LANES)] = zeros16
        return 0

    lax.fori_loop(0, (LANES * HIST) // (8 * LANES), zero_body, 0)

    def scat_body(k, _):
        base = k * (4 * LANES)
        for u in range(4):
            ids16 = ids_v[pl.ds(base + u * LANES, LANES)]
            plsc.addupdate_scatter(hist_v, [ids16 + lane_off], ones16)
        return 0

    lax.fori_loop(0, CHUNK // (4 * LANES), scat_body, 0)

    pltpu.sync_copy(hist_v, out_hbm.at[wid])


def _tc_combine(conf_ref, acc_ref, labmax_ref, out_ref):
    acc = jnp.sum(acc_ref[...], axis=0)
    diff = jnp.abs(conf_ref[...] - acc[:, :C])
    nc = labmax_ref[0, 0] + 1
    valid = lax.broadcasted_iota(jnp.int32, (N_BINS, C), 1) < nc
    total = jnp.sum(jnp.where(valid, diff, 0.0), axis=(0, 1), keepdims=True)
    out_ref[...] = total / (nc.astype(jnp.float32) * float(N))


def kernel(logits, labels):
    labels3 = labels.reshape(NB, R, 1)

    conf, flat, labmax = pl.pallas_call(
        _tc_main,
        grid=(NB,),
        in_specs=[
            pl.BlockSpec((R, C), lambda i: (i, 0)),
            pl.BlockSpec((1, R, 1), lambda i: (i, 0, 0)),
        ],
        out_specs=[
            pl.BlockSpec((N_BINS, C), lambda i: (0, 0)),
            pl.BlockSpec((1, R, 1), lambda i: (i, 0, 0)),
            pl.BlockSpec((1, 1), lambda i: (0, 0)),
        ],
        out_shape=[
            jax.ShapeDtypeStruct((N_BINS, C), jnp.float32),
            jax.ShapeDtypeStruct((NB, R, 1), jnp.int32),
            jax.ShapeDtypeStruct((1, 1), jnp.int32),
        ],
        compiler_params=pltpu.CompilerParams(
            dimension_semantics=("arbitrary",)),
    )(logits, labels3)

    ids = flat.reshape(N)

    sc_call = pl.kernel(
        _sc_hist,
        out_type=jax.ShapeDtypeStruct((NW, LANES * HIST), jnp.float32),
        mesh=plsc.VectorSubcoreMesh(core_axis_name="c", subcore_axis_name="s"),
        scratch_types=[
            pltpu.VMEM((CHUNK,), jnp.int32),
            pltpu.VMEM((LANES * HIST,), jnp.float32),
        ],
    )
    acc_parts = sc_call(ids)
    acc3 = acc_parts.reshape(NW * LANES, N_BINS, 128)

    ece = pl.pallas_call(
        _tc_combine,
        in_specs=[
            pl.BlockSpec((N_BINS, C), lambda: (0, 0)),
            pl.BlockSpec((NW * LANES, N_BINS, 128), lambda: (0, 0, 0)),
            pl.BlockSpec(memory_space=pltpu.SMEM),
        ],
        out_specs=pl.BlockSpec((1, 1), lambda: (0, 0)),
        out_shape=jax.ShapeDtypeStruct((1, 1), jnp.float32),
    )(conf, acc3, labmax)

    return ece[0, 0]


# speedup vs baseline: 140.9902x; 140.9902x over previous
import functools

import jax
import jax.numpy as jnp
from jax import lax
from jax.experimental import pallas as pl
from jax.experimental.pallas import tpu as pltpu
from jax.experimental.pallas import tpu_sc as plsc

N_BINS = 15
N = 262144
C = 100
R = 1024
NB = N // R
HIST = N_BINS * 128
NW = 32
CHUNK = N // NW
LANES = 16


def _tc_main(x_ref, lab_ref, conf_ref, flat_ref, labmax_ref):
    i = pl.program_id(0)
    x = x_ref[...]
    lab = lab_ref[0]

    m = jnp.max(x, axis=1, keepdims=True)
    e = jnp.exp(x - m)
    s = jnp.sum(e, axis=1, keepdims=True)
    sm = e / s

    bi = jnp.clip(jnp.ceil(sm * float(N_BINS)) - 1.0, 0.0, float(N_BINS - 1))

    rows = [
        jnp.sum(jnp.where(bi == float(b), sm, 0.0), axis=0, keepdims=True)
        for b in range(N_BINS)
    ]
    part = jnp.concatenate(rows, axis=0)

    onehot = lax.broadcasted_iota(jnp.int32, (R, C), 1) == lab
    conf_lab = jnp.sum(jnp.where(onehot, sm, 0.0), axis=1, keepdims=True)
    bl = jnp.clip(jnp.ceil(conf_lab * float(N_BINS)) - 1.0, 0.0,
                  float(N_BINS - 1)).astype(jnp.int32)
    flat_ref[0] = bl * 128 + lab

    lm = jnp.max(lab, axis=0, keepdims=True)

    @pl.when(i == 0)
    def _():
        conf_ref[...] = part
        labmax_ref[...] = lm

    @pl.when(i > 0)
    def _():
        conf_ref[...] += part
        labmax_ref[...] = jnp.maximum(labmax_ref[...], lm)


def _sc_hist(ids_hbm, out_hbm, ids_v, hist_v):
    info = plsc.get_sparse_core_info()
    wid = lax.axis_index("s") * info.num_cores + lax.axis_index("c")
    pltpu.sync_copy(ids_hbm.at[pl.ds(wid * CHUNK, CHUNK)], ids_v)

    zeros16 = jnp.zeros((LANES,), jnp.float32)
    ones16 = jnp.ones((LANES,), jnp.float32)
    lane_off = lax.iota(jnp.int32, LANES) * HIST

    def zero_body(j, _):
        base = j * (8 * LANES)
        for u in range(8):
            hist_v[pl.ds(base + u * LANES, LANES)] = zeros16
        return 0

    lax.fori_loop(0, (LANES * HIST) // (8 * LANES), zero_body, 0)

    def scat_body(k, _):
        base = k * (4 * LANES)
        for u in range(4):
            ids16 = ids_v[pl.ds(base + u * LANES, LANES)]
            plsc.addupdate_scatter(hist_v, [ids16 + lane_off], ones16)
        return 0

    lax.fori_loop(0, CHUNK // (4 * LANES), scat_body, 0)

    pltpu.sync_copy(hist_v, out_hbm.at[wid])


def _tc_combine(conf_ref, acc_ref, labmax_ref, out_ref):
    acc = jnp.sum(acc_ref[...], axis=0)
    diff = jnp.abs(conf_ref[...] - acc[:, :C])
    nc = labmax_ref[0, 0] + 1
    valid = lax.broadcasted_iota(jnp.int32, (N_BINS, C), 1) < nc
    total = jnp.sum(jnp.where(valid, diff, 0.0), axis=(0, 1), keepdims=True)
    out_ref[...] = total / (nc.astype(jnp.float32) * float(N))


def kernel(logits, labels):
    labels3 = labels.reshape(NB, R, 1)

    conf, flat, labmax = pl.pallas_call(
        _tc_main,
        grid=(NB,),
        in_specs=[
            pl.BlockSpec((R, C), lambda i: (i, 0)),
            pl.BlockSpec((1, R, 1), lambda i: (i, 0, 0)),
        ],
        out_specs=[
            pl.BlockSpec((N_BINS, C), lambda i: (0, 0)),
            pl.BlockSpec((1, R, 1), lambda i: (i, 0, 0)),
            pl.BlockSpec((1, 1), lambda i: (0, 0)),
        ],
        out_shape=[
            jax.ShapeDtypeStruct((N_BINS, C), jnp.float32),
            jax.ShapeDtypeStruct((NB, R, 1), jnp.int32),
            jax.ShapeDtypeStruct((1, 1), jnp.int32),
        ],
        compiler_params=pltpu.CompilerParams(
            dimension_semantics=("arbitrary",)),
    )(logits, labels3)

    ids = flat.reshape(N)

    sc_call = pl.kernel(
        _sc_hist,
        out_type=jax.ShapeDtypeStruct((NW, LANES * HIST), jnp.float32),
        mesh=plsc.VectorSubcoreMesh(core_axis_name="c", subcore_axis_name="s"),
        scratch_types=[
            pltpu.VMEM((CHUNK,), jnp.int32),
            pltpu.VMEM((LANES * HIST,), jnp.float32),
        ],
        compiler_params=pltpu.CompilerParams(needs_layout_passes=False),
    )
    acc_parts = sc_call(ids)
    acc3 = acc_parts.reshape(NW * LANES, N_BINS, 128)

    ece = pl.pallas_call(
        _tc_combine,
        in_specs=[
            pl.BlockSpec((N_BINS, C), lambda: (0, 0)),
            pl.BlockSpec((NW * LANES, N_BINS, 128), lambda: (0, 0, 0)),
            pl.BlockSpec(memory_space=pltpu.SMEM),
        ],
        out_specs=pl.BlockSpec((1, 1), lambda: (0, 0)),
        out_shape=jax.ShapeDtypeStruct((1, 1), jnp.float32),
    )(conf, acc3, labmax)

    return ece[0, 0]


# speedup vs baseline: 144.7659x; 1.0268x over previous
import functools

import jax
import jax.numpy as jnp
from jax import lax
from jax.experimental import pallas as pl
from jax.experimental.pallas import tpu as pltpu
from jax.experimental.pallas import tpu_sc as plsc

N_BINS = 15
N = 262144
C = 100
R = 2048
NB = N // R
HIST = N_BINS * 128
NW = 32
CHUNK = N // NW
LANES = 16


def _tc_main(x_ref, lab_ref, conf_ref, flat_ref, labmax_ref):
    i = pl.program_id(0)
    x = x_ref[...]
    lab = lab_ref[0]

    m = jnp.max(x, axis=1, keepdims=True)
    e = jnp.exp(x - m)
    s = jnp.sum(e, axis=1, keepdims=True)
    sm = e * (1.0 / s)

    bi = jnp.ceil(sm * float(N_BINS))

    rows = [
        jnp.sum(jnp.where(bi == float(b + 1), sm, 0.0), axis=0, keepdims=True)
        for b in range(N_BINS)
    ]
    part = jnp.concatenate(rows, axis=0)

    onehot = lax.broadcasted_iota(jnp.int32, (R, C), 1) == lab
    conf_lab = jnp.sum(jnp.where(onehot, sm, 0.0), axis=1, keepdims=True)
    bl = jnp.clip(jnp.ceil(conf_lab * float(N_BINS)) - 1.0, 0.0,
                  float(N_BINS - 1)).astype(jnp.int32)
    flat_ref[0] = bl * 128 + lab

    lm = jnp.max(lab, axis=0, keepdims=True)

    @pl.when(i == 0)
    def _():
        conf_ref[...] = part
        labmax_ref[...] = lm

    @pl.when(i > 0)
    def _():
        conf_ref[...] += part
        labmax_ref[...] = jnp.maximum(labmax_ref[...], lm)


def _sc_hist(ids_hbm, out_hbm, ids_v, hist_v):
    info = plsc.get_sparse_core_info()
    wid = lax.axis_index("s") * info.num_cores + lax.axis_index("c")
    pltpu.sync_copy(ids_hbm.at[pl.ds(wid * CHUNK, CHUNK)], ids_v)

    zeros16 = jnp.zeros((LANES,), jnp.float32)
    ones16 = jnp.ones((LANES,), jnp.float32)
    lane_off = lax.iota(jnp.int32, LANES) * HIST

    def zero_body(j, _):
        base = j * (8 * LANES)
        for u in range(8):
            hist_v[pl.ds(base + u * LANES, LANES)] = zeros16
        return 0

    lax.fori_loop(0, (LANES * HIST) // (8 * LANES), zero_body, 0)

    def scat_body(k, _):
        base = k * (4 * LANES)
        for u in range(4):
            ids16 = ids_v[pl.ds(base + u * LANES, LANES)]
            plsc.addupdate_scatter(hist_v, [ids16 + lane_off], ones16)
        return 0

    lax.fori_loop(0, CHUNK // (4 * LANES), scat_body, 0)

    pltpu.sync_copy(hist_v, out_hbm.at[wid])


def _tc_combine(conf_ref, acc_ref, labmax_ref, out_ref):
    acc = jnp.sum(acc_ref[...], axis=0)
    diff = jnp.abs(conf_ref[...] - acc[:, :C])
    nc = labmax_ref[0, 0] + 1
    valid = lax.broadcasted_iota(jnp.int32, (N_BINS, C), 1) < nc
    total = jnp.sum(jnp.where(valid, diff, 0.0), axis=(0, 1), keepdims=True)
    out_ref[...] = total / (nc.astype(jnp.float32) * float(N))


def kernel(logits, labels):
    labels3 = labels.reshape(NB, R, 1)

    conf, flat, labmax = pl.pallas_call(
        _tc_main,
        grid=(NB,),
        in_specs=[
            pl.BlockSpec((R, C), lambda i: (i, 0)),
            pl.BlockSpec((1, R, 1), lambda i: (i, 0, 0)),
        ],
        out_specs=[
            pl.BlockSpec((N_BINS, C), lambda i: (0, 0)),
            pl.BlockSpec((1, R, 1), lambda i: (i, 0, 0)),
            pl.BlockSpec((1, 1), lambda i: (0, 0)),
        ],
        out_shape=[
            jax.ShapeDtypeStruct((N_BINS, C), jnp.float32),
            jax.ShapeDtypeStruct((NB, R, 1), jnp.int32),
            jax.ShapeDtypeStruct((1, 1), jnp.int32),
        ],
        compiler_params=pltpu.CompilerParams(
            dimension_semantics=("arbitrary",)),
    )(logits, labels3)

    ids = flat.reshape(N)

    sc_call = pl.kernel(
        _sc_hist,
        out_type=jax.ShapeDtypeStruct((NW, LANES * HIST), jnp.float32),
        mesh=plsc.VectorSubcoreMesh(core_axis_name="c", subcore_axis_name="s"),
        scratch_types=[
            pltpu.VMEM((CHUNK,), jnp.int32),
            pltpu.VMEM((LANES * HIST,), jnp.float32),
        ],
        compiler_params=pltpu.CompilerParams(needs_layout_passes=False),
    )
    acc_parts = sc_call(ids)
    acc3 = acc_parts.reshape(NW * LANES, N_BINS, 128)

    ece = pl.pallas_call(
        _tc_combine,
        in_specs=[
            pl.BlockSpec((N_BINS, C), lambda: (0, 0)),
            pl.BlockSpec((NW * LANES, N_BINS, 128), lambda: (0, 0, 0)),
            pl.BlockSpec(memory_space=pltpu.SMEM),
        ],
        out_specs=pl.BlockSpec((1, 1), lambda: (0, 0)),
        out_shape=jax.ShapeDtypeStruct((1, 1), jnp.float32),
    )(conf, acc3, labmax)

    return ece[0, 0]


# speedup vs baseline: 148.8958x; 1.0285x over previous
import functools

import jax
import jax.numpy as jnp
from jax import lax
from jax.experimental import pallas as pl
from jax.experimental.pallas import tpu as pltpu
from jax.experimental.pallas import tpu_sc as plsc

N_BINS = 15
N = 262144
C = 100
R = 2048
NB = N // R
HIST = N_BINS * 128
NW = 32
CHUNK = N // NW
LANES = 16


def _tc_main(x_ref, lab_ref, conf_ref, flat_ref, labmax_ref):
    i = pl.program_id(0)
    x = x_ref[...]
    lab = lab_ref[0]

    m = jnp.max(x, axis=1, keepdims=True)
    e = jnp.exp(x - m)
    s = jnp.sum(e, axis=1, keepdims=True)
    sm = e * (1.0 / s)

    bi = jnp.ceil(sm * float(N_BINS))

    rows = [
        jnp.sum(jnp.where(bi == float(b + 1), sm, 0.0), axis=0, keepdims=True)
        for b in range(N_BINS)
    ]
    part = jnp.concatenate(rows, axis=0)

    onehot = lax.broadcasted_iota(jnp.int32, (R, C), 1) == lab
    conf_lab = jnp.sum(jnp.where(onehot, sm, 0.0), axis=1, keepdims=True)
    bl = jnp.clip(jnp.ceil(conf_lab * float(N_BINS)) - 1.0, 0.0,
                  float(N_BINS - 1)).astype(jnp.int32)
    fl = bl * 128 + lab
    flat_ref[0] = fl.T

    lm = jnp.max(lab, axis=0, keepdims=True)

    @pl.when(i == 0)
    def _():
        conf_ref[...] = part
        labmax_ref[...] = lm

    @pl.when(i > 0)
    def _():
        conf_ref[...] += part
        labmax_ref[...] = jnp.maximum(labmax_ref[...], lm)


def _sc_hist(ids_hbm, out_hbm, ids_v, hist_v):
    info = plsc.get_sparse_core_info()
    wid = lax.axis_index("s") * info.num_cores + lax.axis_index("c")
    pltpu.sync_copy(ids_hbm.at[pl.ds(wid * CHUNK, CHUNK)], ids_v)

    zeros16 = jnp.zeros((LANES,), jnp.float32)
    ones16 = jnp.ones((LANES,), jnp.float32)
    lane_off = lax.iota(jnp.int32, LANES) * HIST

    def zero_body(j, _):
        base = j * (8 * LANES)
        for u in range(8):
            hist_v[pl.ds(base + u * LANES, LANES)] = zeros16
        return 0

    lax.fori_loop(0, (LANES * HIST) // (8 * LANES), zero_body, 0)

    def scat_body(k, _):
        base = k * (4 * LANES)
        for u in range(4):
            ids16 = ids_v[pl.ds(base + u * LANES, LANES)]
            plsc.addupdate_scatter(hist_v, [ids16 + lane_off], ones16)
        return 0

    lax.fori_loop(0, CHUNK // (4 * LANES), scat_body, 0)

    pltpu.sync_copy(hist_v, out_hbm.at[wid])


def _tc_combine(conf_ref, acc_ref, labmax_ref, out_ref):
    acc = jnp.sum(acc_ref[...], axis=0)
    diff = jnp.abs(conf_ref[...] - acc[:, :C])
    nc = labmax_ref[0, 0] + 1
    valid = lax.broadcasted_iota(jnp.int32, (N_BINS, C), 1) < nc
    total = jnp.sum(jnp.where(valid, diff, 0.0), axis=(0, 1), keepdims=True)
    out_ref[...] = total / (nc.astype(jnp.float32) * float(N))


def kernel(logits, labels):
    labels3 = labels.reshape(NB, R, 1)

    conf, flat, labmax = pl.pallas_call(
        _tc_main,
        grid=(NB,),
        in_specs=[
            pl.BlockSpec((R, C), lambda i: (i, 0)),
            pl.BlockSpec((1, R, 1), lambda i: (i, 0, 0)),
        ],
        out_specs=[
            pl.BlockSpec((N_BINS, C), lambda i: (0, 0)),
            pl.BlockSpec((1, 1, R), lambda i: (i, 0, 0)),
            pl.BlockSpec((1, 1), lambda i: (0, 0)),
        ],
        out_shape=[
            jax.ShapeDtypeStruct((N_BINS, C), jnp.float32),
            jax.ShapeDtypeStruct((NB, 1, R), jnp.int32),
            jax.ShapeDtypeStruct((1, 1), jnp.int32),
        ],
        compiler_params=pltpu.CompilerParams(
            dimension_semantics=("arbitrary",)),
    )(logits, labels3)

    ids = flat.reshape(N)

    sc_call = pl.kernel(
        _sc_hist,
        out_type=jax.ShapeDtypeStruct((NW, LANES * HIST), jnp.float32),
        mesh=plsc.VectorSubcoreMesh(core_axis_name="c", subcore_axis_name="s"),
        scratch_types=[
            pltpu.VMEM((CHUNK,), jnp.int32),
            pltpu.VMEM((LANES * HIST,), jnp.float32),
        ],
        compiler_params=pltpu.CompilerParams(needs_layout_passes=False),
    )
    acc_parts = sc_call(ids)
    acc3 = acc_parts.reshape(NW * LANES, N_BINS, 128)

    ece = pl.pallas_call(
        _tc_combine,
        in_specs=[
            pl.BlockSpec((N_BINS, C), lambda: (0, 0)),
            pl.BlockSpec((NW * LANES, N_BINS, 128), lambda: (0, 0, 0)),
            pl.BlockSpec(memory_space=pltpu.SMEM),
        ],
        out_specs=pl.BlockSpec((1, 1), lambda: (0, 0)),
        out_shape=jax.ShapeDtypeStruct((1, 1), jnp.float32),
    )(conf, acc3, labmax)

    return ece[0, 0]
